# baseline (speedup 1.0000x reference)
import functools

import numpy as np
import jax
import jax.numpy as jnp
from jax import lax
from jax.experimental import pallas as pl
from jax.experimental.pallas import tpu as pltpu
from jax.experimental.pallas import tpu_sc as plsc

NC = 2
NS = 16
NW = NC * NS
D = 1024
C = 32
NBUF = 3

K_HI = 72
K_LO = 128
R = 512


@functools.lru_cache(maxsize=None)
def _make_sc(B):
    bpw = B // NW
    nchunks = bpw // C
    mesh = plsc.VectorSubcoreMesh(core_axis_name="c", subcore_axis_name="s")

    @functools.partial(
        pl.kernel,
        mesh=mesh,
        out_type=jax.ShapeDtypeStruct((B, D), jnp.float32),
        scratch_types=[
            pltpu.VMEM((bpw,), jnp.int32),
            pltpu.VMEM((NBUF, C, D), jnp.float32),
            pltpu.SemaphoreType.DMA,
            pltpu.SemaphoreType.DMA,
        ],
    )
    def gather_kernel(table_hbm, idx_hbm, out_hbm, idx_v, rows_v, gsem, ssem):
        wid = lax.axis_index("s") * NC + lax.axis_index("c")
        base = wid * bpw
        pltpu.sync_copy(idx_hbm.at[pl.ds(base, bpw)], idx_v)

        def gather(i):
            return pltpu.make_async_copy(
                table_hbm.at[idx_v.at[pl.ds(i * C, C)]],
                rows_v.at[lax.rem(i, NBUF)],
                gsem,
            )

        def writeback(i):
            return pltpu.make_async_copy(
                rows_v.at[lax.rem(i, NBUF)],
                out_hbm.at[pl.ds(base + i * C, C)],
                ssem,
            )

        for j in range(NBUF - 1):
            gather(j).start()

        def chunk(i, carry):
            @pl.when(i + NBUF - 1 < nchunks)
            def _prefetch():
                @pl.when(i >= 1)
                def _():
                    writeback(i - 1).wait()

                gather(i + NBUF - 1).start()

            gather(i).wait()
            writeback(i).start()
            return carry

        lax.fori_loop(0, nchunks, chunk, 0, unroll=False)
        for j in range(NBUF):
            writeback(nchunks - NBUF + j).wait()

    return gather_kernel


@functools.lru_cache(maxsize=None)
def _trig_tables():
    j = np.arange(D, dtype=np.float64)
    f = np.power(10000.0, -2.0 * np.floor(j / 2.0) / D)
    p = np.where(j % 2 == 1, np.pi / 2.0, 0.0)
    h = np.arange(K_HI, dtype=np.float64)[:, None]
    SH = np.sin(h * 128.0 * f[None, :])
    CH = np.cos(h * 128.0 * f[None, :])
    l = np.arange(K_LO, dtype=np.float64)[:, None]
    SL = np.sin(l * f[None, :] + p[None, :])
    CL = np.cos(l * f[None, :] + p[None, :])
    TH = np.concatenate([SH, CH], axis=1)
    TL = np.concatenate([CL, SL], axis=1)
    return jnp.asarray(TH, jnp.bfloat16), jnp.asarray(TL, jnp.bfloat16)


def _tc_body(idx_ref, th_ref, tl_ref, out_ref):
    idxc = idx_ref[0]
    hi = idxc >> 7
    lo = idxc & 127
    oh_hi = (hi == lax.broadcasted_iota(jnp.int32, (R, K_HI), 1)).astype(jnp.bfloat16)
    oh_lo = (lo == lax.broadcasted_iota(jnp.int32, (R, K_LO), 1)).astype(jnp.bfloat16)
    g_hi = jnp.dot(oh_hi, th_ref[...], preferred_element_type=jnp.float32)
    g_lo = jnp.dot(oh_lo, tl_ref[...], preferred_element_type=jnp.float32)
    res = g_hi[:, :D] * g_lo[:, :D] + g_hi[:, D:] * g_lo[:, D:]
    out_ref[0] = jnp.where(idxc != 0, res, 0.0)


def _tc_synth(idx_flat):
    B = idx_flat.shape[0]
    nblocks = B // R
    th, tl = _trig_tables()
    idx3 = idx_flat.reshape(nblocks, R, 1)
    return pl.pallas_call(
        _tc_body,
        grid=(nblocks,),
        in_specs=[
            pl.BlockSpec((1, R, 1), lambda i: (i, 0, 0)),
            pl.BlockSpec((K_HI, 2 * D), lambda i: (0, 0)),
            pl.BlockSpec((K_LO, 2 * D), lambda i: (0, 0)),
        ],
        out_specs=pl.BlockSpec((1, R, D), lambda i: (i, 0, 0)),
        out_shape=jax.ShapeDtypeStruct((nblocks, R, D), jnp.float32),
    )(idx3, th, tl).reshape(B, D)


def kernel(input_batch, table):
    shape = input_batch.shape
    idx = input_batch.reshape(-1).astype(jnp.int32)
    B = idx.shape[0]
    S = B // 2
    sc_part = _make_sc(S)(table, idx[:S])
    tc_part = _tc_synth(idx[S:])
    out = jnp.concatenate([sc_part, tc_part], axis=0)
    return out.reshape(*shape, D)


# speedup vs baseline: 1.8659x; 1.8659x over previous
import functools

import jax
import jax.numpy as jnp
from jax import lax
from jax.experimental import pallas as pl
from jax.experimental.pallas import tpu as pltpu
from jax.experimental.pallas import tpu_sc as plsc

NC = 2
NS = 16
NW = NC * NS
D = 1024
C = 16
NBUF = 6


@functools.lru_cache(maxsize=None)
def _make(B):
    bpw = B // NW
    nchunks = bpw // C
    mesh = plsc.VectorSubcoreMesh(core_axis_name="c", subcore_axis_name="s")

    @functools.partial(
        pl.kernel,
        mesh=mesh,
        out_type=jax.ShapeDtypeStruct((B, D), jnp.float32),
        scratch_types=[
            pltpu.VMEM((bpw,), jnp.int32),
            pltpu.VMEM((NBUF, C, D), jnp.float32),
            pltpu.SemaphoreType.DMA,
            pltpu.SemaphoreType.DMA,
        ],
    )
    def gather_kernel(table_hbm, idx_hbm, out_hbm, idx_v, rows_v, gsem, ssem):
        wid = lax.axis_index("s") * NC + lax.axis_index("c")
        base = wid * bpw
        pltpu.sync_copy(idx_hbm.at[pl.ds(base, bpw)], idx_v)

        def gather(i):
            return pltpu.make_async_copy(
                table_hbm.at[idx_v.at[pl.ds(i * C, C)]],
                rows_v.at[lax.rem(i, NBUF)],
                gsem,
            )

        def writeback(i):
            return pltpu.make_async_copy(
                rows_v.at[lax.rem(i, NBUF)],
                out_hbm.at[pl.ds(base + i * C, C)],
                ssem,
            )

        for j in range(NBUF - 1):
            gather(j).start()

        def chunk(i, carry):
            @pl.when(i + NBUF - 1 < nchunks)
            def _prefetch():
                @pl.when(i >= 1)
                def _():
                    writeback(i - 1).wait()

                gather(i + NBUF - 1).start()

            gather(i).wait()
            writeback(i).start()
            return carry

        lax.fori_loop(0, nchunks, chunk, 0, unroll=False)
        for j in range(NBUF):
            writeback(nchunks - NBUF + j).wait()

    return gather_kernel


def kernel(input_batch, table):
    shape = input_batch.shape
    idx = input_batch.reshape(-1).astype(jnp.int32)
    out = _make(idx.shape[0])(table, idx)
    return out.reshape(*shape, D)


# speedup vs baseline: 1.8681x; 1.0012x over previous
import functools

import jax
import jax.numpy as jnp
from jax import lax
from jax.experimental import pallas as pl
from jax.experimental.pallas import tpu as pltpu
from jax.experimental.pallas import tpu_sc as plsc

NC = 2
NS = 16
NW = NC * NS
D = 1024
C = 32
NBUF = 3


@functools.lru_cache(maxsize=None)
def _make(B):
    bpw = B // NW
    nchunks = bpw // C
    mesh = plsc.VectorSubcoreMesh(core_axis_name="c", subcore_axis_name="s")

    @functools.partial(
        pl.kernel,
        mesh=mesh,
        out_type=jax.ShapeDtypeStruct((B, D), jnp.float32),
        scratch_types=[
            pltpu.VMEM((bpw,), jnp.int32),
            pltpu.VMEM((NBUF, C, D), jnp.float32),
            pltpu.SemaphoreType.DMA,
            pltpu.SemaphoreType.DMA,
        ],
    )
    def gather_kernel(table_hbm, idx_hbm, out_hbm, idx_v, rows_v, gsem, ssem):
        wid = lax.axis_index("s") * NC + lax.axis_index("c")
        base = wid * bpw
        pltpu.sync_copy(idx_hbm.at[pl.ds(base, bpw)], idx_v)

        def gather(i):
            return pltpu.make_async_copy(
                table_hbm.at[idx_v.at[pl.ds(i * C, C)]],
                rows_v.at[lax.rem(i, NBUF)],
                gsem,
            )

        def writeback(i):
            return pltpu.make_async_copy(
                rows_v.at[lax.rem(i, NBUF)],
                out_hbm.at[pl.ds(base + i * C, C)],
                ssem,
            )

        for j in range(NBUF - 1):
            gather(j).start()

        def chunk(i, carry):
            @pl.when(i + NBUF - 1 < nchunks)
            def _prefetch():
                @pl.when(i >= 1)
                def _():
                    writeback(i - 1).wait()

                gather(i + NBUF - 1).start()

            gather(i).wait()
            writeback(i).start()
            return carry

        lax.fori_loop(0, nchunks, chunk, 0, unroll=False)
        for j in range(NBUF):
            writeback(nchunks - NBUF + j).wait()

    return gather_kernel


def kernel(input_batch, table):
    shape = input_batch.shape
    idx = input_batch.reshape(-1).astype(jnp.int32)
    out = _make(idx.shape[0])(table, idx)
    return out.reshape(*shape, D)
